# baseline (speedup 1.0000x reference)
import functools

import jax
import jax.numpy as jnp
from jax import lax
from jax.experimental import pallas as pl
from jax.experimental.pallas import tpu as pltpu
from jax.experimental.pallas import tpu_sc as plsc

N_VOCAB = 1000000
EMB_DIM = 64
BATCH = 4096
SEQ_LEN = 200

NC = 2
NS = 16
PLANES_PER_SC = EMB_DIM // NC
B_PER_TEC = BATCH // NS
CH_S = 25
NCH = SEQ_LEN // CH_S

LOAD_CH = 62496
LOAD_TAIL = N_VOCAB - 15 * LOAD_CH

VC = 124928
NVC = 9
TAIL_V = N_VOCAB - 8 * VC


def _detile_body(in_ref, out_ref):
    for f in range(8):
        out_ref[pl.ds(f * VC, VC)] = in_ref[f]


_detile = pl.pallas_call(
    _detile_body,
    grid=(8, NVC),
    in_specs=[pl.BlockSpec((8, VC), lambda i, j: (i, j))],
    out_specs=pl.BlockSpec((8 * VC, ), lambda i, j: (j * 8 + i,)),
    out_shape=jax.ShapeDtypeStruct((NVC * EMB_DIM * VC,), jnp.float32),
)


def _make_gather():
    mesh = plsc.VectorSubcoreMesh(core_axis_name="c", subcore_axis_name="s")

    @functools.partial(
        pl.kernel,
        mesh=mesh,
        compiler_params=pltpu.CompilerParams(use_tc_tiling_on_sc=False),
        out_type=jax.ShapeDtypeStruct((SEQ_LEN * 8 * 32 * 8 * 128,), jnp.float32),
        scratch_types=[
            pltpu.VMEM_SHARED((N_VOCAB,), jnp.float32),
            pltpu.VMEM_SHARED((2, CH_S * 128), jnp.float32),
            pltpu.VMEM((2, SEQ_LEN * 128), jnp.int32),
            pltpu.VMEM((2, 2, CH_S * 128), jnp.float32),
            pltpu.VMEM((CH_S * 128,), jnp.int32),
            pltpu.SemaphoreType.DMA,
            pltpu.SemaphoreType.DMA,
            pltpu.SemaphoreType.DMA,
        ],
    )
    def gather_kernel(idxT_hbm, tableT_hbm, out_hbm, plane_smem, dummy_sh,
                      idx_v, dstbuf, tmpl, lsem, gsem, wsem):
        cid = lax.axis_index("c")
        sid = lax.axis_index("s")
        b0 = sid * B_PER_TEC
        off = sid * LOAD_CH

        VCS = VC // NS

        def load(p):
            def lfire(j, carry):
                pltpu.async_copy(
                    tableT_hbm.at[pl.ds((j * EMB_DIM + p) * VC
                                        + sid * VCS, VCS)],
                    plane_smem.at[pl.ds(j * VC + sid * VCS, VCS)], lsem)
                return carry

            lax.fori_loop(0, 8, lfire, 0)

            @pl.when(sid == 0)
            def _():
                pltpu.async_copy(
                    tableT_hbm.at[pl.ds((8 * EMB_DIM + p) * VC, TAIL_V)],
                    plane_smem.at[pl.ds(8 * VC, TAIL_V)], lsem)

        def wait_my_load():
            def lwait(j, carry):
                pltpu.make_async_copy(
                    tableT_hbm.at[pl.ds(0, VCS)],
                    plane_smem.at[pl.ds(0, VCS)], lsem).wait()
                return carry

            lax.fori_loop(0, 8, lwait, 0)

            @pl.when(sid == 0)
            def _():
                pltpu.make_async_copy(
                    tableT_hbm.at[pl.ds(0, TAIL_V)],
                    plane_smem.at[pl.ds(0, TAIL_V)], lsem).wait()

        def drain(sem):
            pltpu.make_async_copy(dstbuf.at[0], dummy_sh, sem).wait()

        def tfire(k, carry):
            base = (k // 8) * 262144 + (k % 8) * 16
            tmpl[pl.ds(k * 16, 16)] = base + lax.iota(jnp.int32, 16)
            return carry

        lax.fori_loop(0, CH_S * 8, tfire, 0)

        def ifire(s_, carry):
            for h2 in (0, 1):
                pltpu.async_copy(
                    idxT_hbm.at[pl.ds(s_ * BATCH + b0 + h2 * 128, 128)],
                    idx_v.at[h2, pl.ds(s_ * 128, 128)], gsem)
            return carry

        lax.fori_loop(0, SEQ_LEN, ifire, 0)

        def idrain(c, carry):
            drain(gsem)
            return carry

        lax.fori_loop(0, NCH, idrain, 0)

        SPAN = (CH_S - 1) * 262144 + 128

        def do_chunk(j, c):
            p = cid * PLANES_PER_SC + j
            a = p // 8
            fi = p % 8
            buf = lax.rem(c, 2)

            @pl.when(j * NCH + c >= 2)
            def _():
                drain(wsem)

            for h2 in (0, 1):
                pltpu.async_copy(
                    plane_smem.at[idx_v.at[h2, pl.ds(c * CH_S * 128,
                                                     CH_S * 128)]],
                    dstbuf.at[buf, h2], gsem)
            drain(gsem)
            for h2 in (0, 1):
                base = (c * CH_S) * 262144 + a * 32768 \
                    + (2 * sid + h2) * 1024 + fi * 128
                pltpu.async_copy(
                    dstbuf.at[buf, h2],
                    out_hbm.at[pl.ds(base, SPAN)].at[tmpl],
                    wsem)

        def plane_body(j, carry):
            load(cid * PLANES_PER_SC + j)
            wait_my_load()
            plsc.subcore_barrier()

            def chunk_body(c, carry2):
                do_chunk(j, c)
                return carry2

            lax.fori_loop(0, NCH, chunk_body, 0)
            plsc.subcore_barrier()
            return carry

        lax.fori_loop(0, PLANES_PER_SC, plane_body, 0)
        drain(wsem)
        drain(wsem)

    return gather_kernel


_gather = _make_gather()


def kernel(word_seq, emb_weight):
    idxT = word_seq.T.reshape(SEQ_LEN * BATCH)
    table_lin = _detile(emb_weight.T)
    out6 = _gather(idxT, table_lin).reshape(SEQ_LEN, 8, 32, 8, 128)
    return out6.transpose(2, 4, 0, 1, 3).reshape(BATCH, SEQ_LEN, EMB_DIM)


# speedup vs baseline: 132.6169x; 132.6169x over previous
import functools

import jax
import jax.numpy as jnp
from jax import lax
from jax.experimental import pallas as pl
from jax.experimental.pallas import tpu as pltpu
from jax.experimental.pallas import tpu_sc as plsc

N_VOCAB = 1000000
EMB_DIM = 64
BATCH = 4096
SEQ_LEN = 200

NC = 2
NS = 16
PLANES_PER_SC = EMB_DIM // NC
B_PER_TEC = BATCH // NS
CH_S = 25
NCH = SEQ_LEN // CH_S

LOAD_CH = 62496
LOAD_TAIL = N_VOCAB - 15 * LOAD_CH

VC = 124928
NVC = 9
TAIL_V = N_VOCAB - 8 * VC


def _detile_body(in_ref, out_ref):
    for f in range(8):
        out_ref[pl.ds(f * VC, VC)] = in_ref[f]


_detile = pl.pallas_call(
    _detile_body,
    grid=(8, NVC),
    in_specs=[pl.BlockSpec((8, VC), lambda i, j: (i, j))],
    out_specs=pl.BlockSpec((8 * VC, ), lambda i, j: (j * 8 + i,)),
    out_shape=jax.ShapeDtypeStruct((NVC * EMB_DIM * VC,), jnp.float32),
)


def _make_gather():
    mesh = plsc.VectorSubcoreMesh(core_axis_name="c", subcore_axis_name="s")

    @functools.partial(
        pl.kernel,
        mesh=mesh,
        compiler_params=pltpu.CompilerParams(use_tc_tiling_on_sc=False),
        out_type=jax.ShapeDtypeStruct((SEQ_LEN, 8, 32, 8, 128), jnp.float32),
        scratch_types=[
            pltpu.VMEM_SHARED((N_VOCAB,), jnp.float32),
            pltpu.VMEM_SHARED((CH_S, 2, 128), jnp.float32),
            pltpu.VMEM((SEQ_LEN, B_PER_TEC), jnp.int32),
            pltpu.VMEM((2, CH_S, 2, 128), jnp.float32),
            pltpu.SemaphoreType.DMA,
            pltpu.SemaphoreType.DMA,
            pltpu.SemaphoreType.DMA,
        ],
    )
    def gather_kernel(idxT_hbm, tableT_hbm, out_hbm, plane_smem, dummy_sh,
                      idx_v, dstbuf, lsem, gsem, wsem):
        cid = lax.axis_index("c")
        sid = lax.axis_index("s")
        b0 = sid * B_PER_TEC
        off = sid * LOAD_CH

        VCS = VC // NS

        def load(p):
            def lfire(j, carry):
                pltpu.async_copy(
                    tableT_hbm.at[pl.ds((j * EMB_DIM + p) * VC
                                        + sid * VCS, VCS)],
                    plane_smem.at[pl.ds(j * VC + sid * VCS, VCS)], lsem)
                return carry

            lax.fori_loop(0, 8, lfire, 0)

            @pl.when(sid == 0)
            def _():
                pltpu.async_copy(
                    tableT_hbm.at[pl.ds((8 * EMB_DIM + p) * VC, TAIL_V)],
                    plane_smem.at[pl.ds(8 * VC, TAIL_V)], lsem)

        def wait_my_load():
            def lwait(j, carry):
                pltpu.make_async_copy(
                    tableT_hbm.at[pl.ds(0, VCS)],
                    plane_smem.at[pl.ds(0, VCS)], lsem).wait()
                return carry

            lax.fori_loop(0, 8, lwait, 0)

            @pl.when(sid == 0)
            def _():
                pltpu.make_async_copy(
                    tableT_hbm.at[pl.ds(0, TAIL_V)],
                    plane_smem.at[pl.ds(0, TAIL_V)], lsem).wait()

        def drain(sem):
            pltpu.make_async_copy(dstbuf.at[0], dummy_sh, sem).wait()

        def ifire(s_, carry):
            pltpu.async_copy(idxT_hbm.at[pl.ds(s_ * BATCH + b0, B_PER_TEC)],
                             idx_v.at[s_], gsem)
            return carry

        lax.fori_loop(0, SEQ_LEN, ifire, 0)

        def idrain(c, carry):
            drain(gsem)
            return carry

        lax.fori_loop(0, NCH, idrain, 0)

        def do_chunk(j, c):
            p = cid * PLANES_PER_SC + j
            a = p // 8
            fi = p % 8
            buf = lax.rem(c, 2)

            @pl.when(j * NCH + c >= 2)
            def _():
                drain(wsem)

            def gfire(s_local, carry):
                s_ = c * CH_S + s_local
                for h2 in (0, 1):
                    pltpu.async_copy(
                        plane_smem.at[idx_v.at[s_, pl.ds(h2 * 128, 128)]],
                        dstbuf.at[buf, s_local, h2], gsem)
                return carry

            lax.fori_loop(0, CH_S, gfire, 0)
            drain(gsem)
            pltpu.async_copy(
                dstbuf.at[buf],
                out_hbm.at[pl.ds(c * CH_S, CH_S), a,
                           pl.ds(2 * sid, 2), fi, :],
                wsem)

        def plane_body(j, carry):
            load(cid * PLANES_PER_SC + j)
            wait_my_load()
            plsc.subcore_barrier()

            def chunk_body(c, carry2):
                do_chunk(j, c)
                return carry2

            lax.fori_loop(0, NCH, chunk_body, 0)
            plsc.subcore_barrier()
            return carry

        lax.fori_loop(0, PLANES_PER_SC, plane_body, 0)
        drain(wsem)
        drain(wsem)

    return gather_kernel


_gather = _make_gather()


def kernel(word_seq, emb_weight):
    idxT = word_seq.T.reshape(SEQ_LEN * BATCH)
    table_lin = _detile(emb_weight.T)
    out6 = _gather(idxT, table_lin)
    return out6.transpose(2, 4, 0, 1, 3).reshape(BATCH, SEQ_LEN, EMB_DIM)
